# baseline (speedup 1.0000x reference)
import functools

import jax
import jax.numpy as jnp
from jax import lax
from jax.experimental import pallas as pl
from jax.experimental.pallas import tpu as pltpu
from jax.experimental.pallas import tpu_sc as plsc

NNODES = 50000
NPAD = 50176
EPAD = 819200
EROWS = EPAD // 128
NC, NS = 2, 16
HALF = NPAD // 2
LPAD = 25600
LDUMMY = HALF + 16
ZC = 320
DUMMY = NNODES
KB = 8

_MESH = plsc.VectorSubcoreMesh(
    core_axis_name="c", subcore_axis_name="s", num_cores=NC, num_subcores=NS
)


def _make_deg():
    def body(dst_hbm, zeros_hbm, out_hbm, dst_v, ones_v, bounce_v, deg_sh, sem):
        c = lax.axis_index("c")
        s = lax.axis_index("s")
        stripe = NPAD // NS
        for j in range(128 // 16):
            ones_v[pl.ds(j * 16, 16)] = jnp.ones((16,), jnp.float32)
        pltpu.sync_copy(zeros_hbm, bounce_v)
        pltpu.sync_copy(bounce_v, deg_sh.at[pl.ds(s * stripe, stripe)])
        plsc.subcore_barrier()
        rows_w = EROWS // (NC * NS)
        base = c * (EROWS // NC) + s * rows_w

        def chunk(b, carry):
            pltpu.sync_copy(dst_hbm.at[pl.ds(base + b * KB, KB)], dst_v)
            for r in range(KB):
                pltpu.sync_copy(ones_v, deg_sh.at[dst_v.at[r]], add=True)
            return carry

        lax.fori_loop(0, rows_w // KB, chunk, 0)
        plsc.subcore_barrier()
        pltpu.sync_copy(deg_sh.at[pl.ds(s * stripe, stripe)], bounce_v)
        pltpu.sync_copy(bounce_v, out_hbm.at[pl.ds(c * NPAD + s * stripe, stripe)])

    return pl.kernel(
        body,
        out_type=jax.ShapeDtypeStruct((NC * NPAD,), jnp.float32),
        mesh=_MESH,
        compiler_params=pltpu.CompilerParams(use_tc_tiling_on_sc=False),
        scratch_types=[
            pltpu.VMEM((KB, 128), jnp.int32),
            pltpu.VMEM((128,), jnp.float32),
            pltpu.VMEM((NPAD // NS,), jnp.float32),
            pltpu.VMEM_SHARED((NPAD,), jnp.float32),
            pltpu.SemaphoreType.DMA,
        ],
    )


def _make_aggA():
    def body(src_hbm, dst_hbm, g3_hbm, zeros_hbm, out_hbm, *rest):
        src_v, dst_v, loc_v = rest[0:3]
        rows = rest[3:3 + KB]
        bounce_v = rest[3 + KB]
        agg_sh = rest[4 + KB]
        sems = rest[5 + KB:5 + 2 * KB]
        c = lax.axis_index("c")
        s = lax.axis_index("s")
        zstripe = LPAD // NS
        fstripe = HALF // NS
        pltpu.sync_copy(zeros_hbm, bounce_v)
        pltpu.sync_copy(bounce_v, agg_sh.at[pl.ds(s * zstripe, zstripe)])
        plsc.subcore_barrier()
        rows_w = EROWS // NS
        base = s * rows_w
        offset = c * HALF

        def chunk(b, carry):
            pltpu.sync_copy(src_hbm.at[pl.ds(base + b * KB, KB)], src_v)
            pltpu.sync_copy(dst_hbm.at[pl.ds(base + b * KB, KB)], dst_v)
            descs = [
                pltpu.async_copy(g3_hbm.at[src_v.at[r]], rows[r], sems[r])
                for r in range(KB)
            ]
            for r in range(KB):
                for k in range(128 // 16):
                    v = dst_v[r, pl.ds(k * 16, 16)]
                    loc = v - offset
                    oob = (loc < 0) | (loc >= HALF)
                    loc_v[r, pl.ds(k * 16, 16)] = jnp.where(oob, LDUMMY, loc)
            for r in range(KB):
                descs[r].wait()
                pltpu.sync_copy(rows[r], agg_sh.at[loc_v.at[r]], add=True)
            return carry

        lax.fori_loop(0, rows_w // KB, chunk, 0)
        plsc.subcore_barrier()
        pltpu.sync_copy(agg_sh.at[pl.ds(s * fstripe, fstripe)],
                        bounce_v.at[pl.ds(0, fstripe)])
        pltpu.sync_copy(bounce_v.at[pl.ds(0, fstripe)],
                        out_hbm.at[pl.ds(c * HALF + s * fstripe, fstripe)])

    return pl.kernel(
        body,
        out_type=jax.ShapeDtypeStruct((NPAD, 8), jnp.float32),
        mesh=_MESH,
        compiler_params=pltpu.CompilerParams(use_tc_tiling_on_sc=False),
        scratch_types=(
            [pltpu.VMEM((KB, 128), jnp.int32)] * 3
            + [pltpu.VMEM((128, 8), jnp.float32) for _ in range(KB)]
            + [pltpu.VMEM((LPAD // NS, 8), jnp.float32),
               pltpu.VMEM_SHARED((LPAD, 8), jnp.float32)]
            + [pltpu.SemaphoreType.DMA] * KB
        ),
    )


def _make_aggB():
    def body(src_hbm, dst_hbm, g64a_hbm, g64b_hbm, zeros_hbm,
             outa_hbm, outb_hbm, *rest):
        src_v, dst_v, loc_v = rest[0:3]
        rows = rest[3:3 + KB]
        bounce_v = rest[3 + KB]
        agg_sh = rest[4 + KB]
        sems = rest[5 + KB:5 + 2 * KB]
        c = lax.axis_index("c")
        s = lax.axis_index("s")
        zstripe = LPAD // NS
        fstripe = HALF // NS
        rows_w = EROWS // NS
        base = s * rows_w
        offset = c * HALF
        for h in range(2):
            g_hbm = g64a_hbm if h == 0 else g64b_hbm
            o_hbm = outa_hbm if h == 0 else outb_hbm
            pltpu.sync_copy(zeros_hbm, bounce_v)
            for q in range(zstripe // ZC):
                pltpu.sync_copy(bounce_v,
                                agg_sh.at[pl.ds(s * zstripe + q * ZC, ZC)])
            plsc.subcore_barrier()

            def chunk(b, carry):
                pltpu.sync_copy(src_hbm.at[pl.ds(base + b * KB, KB)], src_v)
                pltpu.sync_copy(dst_hbm.at[pl.ds(base + b * KB, KB)], dst_v)
                descs = [
                    pltpu.async_copy(g_hbm.at[src_v.at[r]], rows[r], sems[r])
                    for r in range(KB)
                ]
                for r in range(KB):
                    for k in range(128 // 16):
                        v = dst_v[r, pl.ds(k * 16, 16)]
                        loc = v - offset
                        oob = (loc < 0) | (loc >= HALF)
                        loc_v[r, pl.ds(k * 16, 16)] = jnp.where(
                            oob, LDUMMY, loc)
                for r in range(KB):
                    descs[r].wait()
                    pltpu.sync_copy(rows[r], agg_sh.at[loc_v.at[r]], add=True)
                return carry

            lax.fori_loop(0, rows_w // KB, chunk, 0)
            plsc.subcore_barrier()
            off = 0
            while off < fstripe:
                sz = min(ZC, fstripe - off)
                pltpu.sync_copy(agg_sh.at[pl.ds(s * fstripe + off, sz)],
                                bounce_v.at[pl.ds(0, sz)])
                pltpu.sync_copy(
                    bounce_v.at[pl.ds(0, sz)],
                    o_hbm.at[pl.ds(c * HALF + s * fstripe + off, sz)])
                off += sz
            plsc.subcore_barrier()

    return pl.kernel(
        body,
        out_type=(jax.ShapeDtypeStruct((NPAD, 32), jnp.float32),
                  jax.ShapeDtypeStruct((NPAD, 32), jnp.float32)),
        mesh=_MESH,
        compiler_params=pltpu.CompilerParams(use_tc_tiling_on_sc=False),
        scratch_types=(
            [pltpu.VMEM((KB, 128), jnp.int32)] * 3
            + [pltpu.VMEM((128, 32), jnp.float32) for _ in range(KB)]
            + [pltpu.VMEM((ZC, 32), jnp.float32),
               pltpu.VMEM_SHARED((LPAD, 32), jnp.float32)]
            + [pltpu.SemaphoreType.DMA] * KB
        ),
    )


def _tc1_body(deg_ref, x_ref, g3_ref):
    deg = deg_ref[0] + deg_ref[1] + 1.0
    dinv = lax.rsqrt(deg)
    g3_ref[...] = x_ref[...] * dinv[:, None]


def _tc1(deg2, xpad):
    r = NPAD // 8
    return pl.pallas_call(
        _tc1_body,
        grid=(8,),
        in_specs=[
            pl.BlockSpec((NC, r), lambda i: (0, i)),
            pl.BlockSpec((r, 8), lambda i: (i, 0)),
        ],
        out_specs=pl.BlockSpec((r, 8), lambda i: (i, 0)),
        out_shape=jax.ShapeDtypeStruct((NPAD, 8), jnp.float32),
    )(deg2, xpad)


_B2 = 1024
_G2 = NPAD // _B2


def _tc2_body(agg_ref, g3_ref, w1_ref, b1_ref, w2_ref, outa_ref, outb_ref):
    g3 = g3_ref[...]
    a3 = agg_ref[...] + g3
    dinv = g3[:, 3:4]
    z = a3 * dinv
    h1 = jnp.maximum(
        jnp.dot(z, w1_ref[...], preferred_element_type=jnp.float32)
        + b1_ref[...], 0.0)
    h2 = jnp.dot(h1, w2_ref[...], preferred_element_type=jnp.float32)
    g64 = h2 * dinv
    outa_ref[...] = g64[:, :32]
    outb_ref[...] = g64[:, 32:]


def _tc2(aggA, g3p, W1p, b1r, W2):
    return pl.pallas_call(
        _tc2_body,
        grid=(_G2,),
        in_specs=[
            pl.BlockSpec((_B2, 8), lambda i: (i, 0)),
            pl.BlockSpec((_B2, 8), lambda i: (i, 0)),
            pl.BlockSpec((8, 128), lambda i: (0, 0)),
            pl.BlockSpec((1, 128), lambda i: (0, 0)),
            pl.BlockSpec((128, 64), lambda i: (0, 0)),
        ],
        out_specs=[pl.BlockSpec((_B2, 32), lambda i: (i, 0)),
                   pl.BlockSpec((_B2, 32), lambda i: (i, 0))],
        out_shape=[jax.ShapeDtypeStruct((NPAD, 32), jnp.float32),
                   jax.ShapeDtypeStruct((NPAD, 32), jnp.float32)],
    )(aggA, g3p, W1p, b1r, W2)


def _tc3_body(agga_ref, aggb_ref, g64a_ref, g64b_ref, g3_ref, b2_ref,
              wf_ref, bf_ref, out_ref, acc_ref):
    i = pl.program_id(0)

    @pl.when(i == 0)
    def _():
        acc_ref[...] = jnp.zeros_like(acc_ref)
        out_ref[...] = jnp.zeros_like(out_ref)

    dinv = g3_ref[:, 3:4]
    rows = i * _B2 + lax.broadcasted_iota(jnp.int32, (_B2, 1), 0)
    valid = rows < NNODES
    pre_a = dinv * (agga_ref[...] + g64a_ref[...]) + b2_ref[:, :32]
    pre_b = dinv * (aggb_ref[...] + g64b_ref[...]) + b2_ref[:, 32:]
    val_a = jnp.where(valid, jnp.maximum(pre_a, 0.0), 0.0)
    val_b = jnp.where(valid, jnp.maximum(pre_b, 0.0), 0.0)
    acc_ref[:, :32] += jnp.sum(val_a, axis=0, keepdims=True)
    acc_ref[:, 32:] += jnp.sum(val_b, axis=0, keepdims=True)

    @pl.when(i == _G2 - 1)
    def _():
        mean = acc_ref[...] * (1.0 / NNODES)
        res = jnp.dot(mean, wf_ref[...], preferred_element_type=jnp.float32)
        out_ref[...] = jnp.broadcast_to(res + bf_ref[...], (8, 128))


def _tc3(agg64a, agg64b, g64a, g64b, g3p, b2r, Wfp, bfr):
    return pl.pallas_call(
        _tc3_body,
        grid=(_G2,),
        in_specs=[
            pl.BlockSpec((_B2, 32), lambda i: (i, 0)),
            pl.BlockSpec((_B2, 32), lambda i: (i, 0)),
            pl.BlockSpec((_B2, 32), lambda i: (i, 0)),
            pl.BlockSpec((_B2, 32), lambda i: (i, 0)),
            pl.BlockSpec((_B2, 8), lambda i: (i, 0)),
            pl.BlockSpec((1, 64), lambda i: (0, 0)),
            pl.BlockSpec((64, 128), lambda i: (0, 0)),
            pl.BlockSpec((1, 128), lambda i: (0, 0)),
        ],
        out_specs=pl.BlockSpec((8, 128), lambda i: (0, 0)),
        out_shape=jax.ShapeDtypeStruct((8, 128), jnp.float32),
        scratch_shapes=[pltpu.VMEM((1, 64), jnp.float32)],
    )(agg64a, agg64b, g64a, g64b, g3p, b2r, Wfp, bfr)


_sc_deg = _make_deg()
_sc_aggA = _make_aggA()
_sc_aggB = _make_aggB()


@jax.jit
def kernel(x, edge_index, W1, b1, W2, b2, Wf, bf):
    e0 = edge_index.shape[1]
    src = jnp.concatenate(
        [edge_index[0], jnp.full((EPAD - e0,), DUMMY, jnp.int32)]
    ).reshape(EROWS, 128)
    dst = jnp.concatenate(
        [edge_index[1], jnp.full((EPAD - e0,), DUMMY, jnp.int32)]
    ).reshape(EROWS, 128)
    xpad = (
        jnp.zeros((NPAD, 8), jnp.float32)
        .at[:NNODES, :3].set(x)
        .at[:NNODES, 3].set(1.0)
    )
    W1p = jnp.zeros((8, 128), jnp.float32).at[:3].set(W1)
    b1r = b1.reshape(1, 128)
    b2r = b2.reshape(1, 64)
    Wfp = jnp.zeros((64, 128), jnp.float32).at[:, :3].set(Wf)
    bfr = jnp.zeros((1, 128), jnp.float32).at[0, :3].set(bf)
    zD = jnp.zeros((NPAD // NS,), jnp.float32)
    zA = jnp.zeros((LPAD // NS, 8), jnp.float32)
    zB = jnp.zeros((ZC, 32), jnp.float32)

    deg2 = _sc_deg(dst, zD).reshape(NC, NPAD)
    g3p = _tc1(deg2, xpad)
    aggA = _sc_aggA(src, dst, g3p, zA)
    g64a, g64b = _tc2(aggA, g3p, W1p, b1r, W2)
    agg64a, agg64b = _sc_aggB(src, dst, g64a, g64b, zB)
    outb = _tc3(agg64a, agg64b, g64a, g64b, g3p, b2r, Wfp, bfr)
    return outb[0, :3]


# speedup vs baseline: 1.4258x; 1.4258x over previous
import jax
import jax.numpy as jnp
from jax import lax
from jax.experimental import pallas as pl
from jax.experimental.pallas import tpu as pltpu
from jax.experimental.pallas import tpu_sc as plsc

NNODES = 50000
NPAD = 50176
EPAD = 819200
EROWS = EPAD // 128
NC, NS = 2, 16
HALF = NPAD // 2
LPAD = 25600
LDUMMY = HALF + 16
ZCB = 64
DUMMY = NNODES
KB = 8

_MESH = plsc.VectorSubcoreMesh(
    core_axis_name="c", subcore_axis_name="s", num_cores=NC, num_subcores=NS
)
_SC_PARAMS = pltpu.CompilerParams(use_tc_tiling_on_sc=False)


def _make_deg():
    def body(dst_hbm, zeros_hbm, out_hbm, dst_v, ones_v, bounce_v, deg_sh,
             sem):
        c = lax.axis_index("c")
        s = lax.axis_index("s")
        stripe = NPAD // NS
        for j in range(128 // 16):
            ones_v[pl.ds(j * 16, 16)] = jnp.ones((16,), jnp.float32)
        pltpu.sync_copy(zeros_hbm, bounce_v)
        pltpu.sync_copy(bounce_v, deg_sh.at[pl.ds(s * stripe, stripe)])
        plsc.subcore_barrier()
        rows_w = EROWS // (NC * NS)
        base = c * (EROWS // NC) + s * rows_w

        def chunk(b, carry):
            pltpu.sync_copy(dst_hbm.at[pl.ds(base + b * KB, KB)], dst_v)
            for r in range(KB):
                pltpu.sync_copy(ones_v, deg_sh.at[dst_v.at[r]], add=True)
            return carry

        lax.fori_loop(0, rows_w // KB, chunk, 0)
        plsc.subcore_barrier()
        pltpu.sync_copy(deg_sh.at[pl.ds(s * stripe, stripe)], bounce_v)
        pltpu.sync_copy(bounce_v,
                        out_hbm.at[pl.ds(c * NPAD + s * stripe, stripe)])

    return pl.kernel(
        body,
        out_type=jax.ShapeDtypeStruct((NC * NPAD,), jnp.float32),
        mesh=_MESH,
        compiler_params=_SC_PARAMS,
        scratch_types=[
            pltpu.VMEM((KB, 128), jnp.int32),
            pltpu.VMEM((128,), jnp.float32),
            pltpu.VMEM((NPAD // NS,), jnp.float32),
            pltpu.VMEM_SHARED((NPAD,), jnp.float32),
            pltpu.SemaphoreType.DMA,
        ],
    )


def _make_aggA():
    def body(src_hbm, dst_hbm, g3_hbm, zeros_hbm, out_hbm, *rest):
        src_v, dst_v = rest[0:2]
        rows = rest[2:2 + KB]
        bounce_v = rest[2 + KB]
        agg_sh = rest[3 + KB]
        sems = rest[4 + KB:4 + 2 * KB]
        c = lax.axis_index("c")
        s = lax.axis_index("s")
        stripe = NPAD // NS
        pltpu.sync_copy(zeros_hbm, bounce_v)
        pltpu.sync_copy(bounce_v, agg_sh.at[pl.ds(s * stripe, stripe)])
        plsc.subcore_barrier()
        rows_w = EROWS // (NC * NS)
        base = c * (EROWS // NC) + s * rows_w

        def chunk(b, carry):
            pltpu.sync_copy(src_hbm.at[pl.ds(base + b * KB, KB)], src_v)
            pltpu.sync_copy(dst_hbm.at[pl.ds(base + b * KB, KB)], dst_v)
            descs = [
                pltpu.async_copy(g3_hbm.at[src_v.at[r]], rows[r], sems[r])
                for r in range(KB)
            ]
            for r in range(KB):
                descs[r].wait()
                pltpu.sync_copy(rows[r], agg_sh.at[dst_v.at[r]], add=True)
            return carry

        lax.fori_loop(0, rows_w // KB, chunk, 0)
        plsc.subcore_barrier()
        pltpu.sync_copy(agg_sh.at[pl.ds(s * stripe, stripe)], bounce_v)
        pltpu.sync_copy(bounce_v,
                        out_hbm.at[pl.ds(c * NPAD + s * stripe, stripe)])

    return pl.kernel(
        body,
        out_type=jax.ShapeDtypeStruct((NC * NPAD, 8), jnp.float32),
        mesh=_MESH,
        compiler_params=_SC_PARAMS,
        scratch_types=(
            [pltpu.VMEM((KB, 128), jnp.int32)] * 2
            + [pltpu.VMEM((128, 8), jnp.float32) for _ in range(KB)]
            + [pltpu.VMEM((NPAD // NS, 8), jnp.float32),
               pltpu.VMEM_SHARED((NPAD, 8), jnp.float32)]
            + [pltpu.SemaphoreType.DMA] * KB
        ),
    )


def _make_aggB():
    def body(src_hbm, dst_hbm, g64_hbm, zeros_hbm, out_hbm, *rest):
        src_v, dst_v, loc_v = rest[0:3]
        rows = rest[3:5]
        bounce_v = rest[5]
        agg_sh = rest[6]
        sems = rest[7:9]
        c = lax.axis_index("c")
        s = lax.axis_index("s")
        zstripe = LPAD // NS
        fstripe = HALF // NS
        pltpu.sync_copy(zeros_hbm, bounce_v)
        for q in range(zstripe // ZCB):
            pltpu.sync_copy(bounce_v,
                            agg_sh.at[pl.ds(s * zstripe + q * ZCB, ZCB)])
        plsc.subcore_barrier()
        rows_w = EROWS // NS
        base = s * rows_w
        offset = c * HALF

        def chunk(b, carry):
            pltpu.sync_copy(src_hbm.at[pl.ds(base + b * KB, KB)], src_v)
            pltpu.sync_copy(dst_hbm.at[pl.ds(base + b * KB, KB)], dst_v)
            d = {
                0: pltpu.async_copy(g64_hbm.at[src_v.at[0]], rows[0],
                                    sems[0]),
                1: pltpu.async_copy(g64_hbm.at[src_v.at[1]], rows[1],
                                    sems[1]),
            }
            for r in range(KB):
                for k in range(128 // 16):
                    v = dst_v[r, pl.ds(k * 16, 16)]
                    loc = v - offset
                    oob = (loc < 0) | (loc >= HALF)
                    loc_v[r, pl.ds(k * 16, 16)] = jnp.where(oob, LDUMMY, loc)
            for r in range(KB):
                d[r].wait()
                pltpu.sync_copy(rows[r % 2], agg_sh.at[loc_v.at[r]], add=True)
                if r + 2 < KB:
                    d[r + 2] = pltpu.async_copy(
                        g64_hbm.at[src_v.at[r + 2]], rows[r % 2],
                        sems[r % 2])
            return carry

        lax.fori_loop(0, rows_w // KB, chunk, 0)
        plsc.subcore_barrier()
        off = 0
        while off < fstripe:
            sz = min(ZCB, fstripe - off)
            pltpu.sync_copy(agg_sh.at[pl.ds(s * fstripe + off, sz)],
                            bounce_v.at[pl.ds(0, sz)])
            pltpu.sync_copy(
                bounce_v.at[pl.ds(0, sz)],
                out_hbm.at[pl.ds(c * HALF + s * fstripe + off, sz)])
            off += sz

    return pl.kernel(
        body,
        out_type=jax.ShapeDtypeStruct((NPAD, 64), jnp.float32),
        mesh=_MESH,
        compiler_params=_SC_PARAMS,
        scratch_types=(
            [pltpu.VMEM((KB, 128), jnp.int32)] * 3
            + [pltpu.VMEM((128, 64), jnp.float32) for _ in range(2)]
            + [pltpu.VMEM((ZCB, 64), jnp.float32),
               pltpu.VMEM_SHARED((LPAD, 64), jnp.float32)]
            + [pltpu.SemaphoreType.DMA] * 2
        ),
    )


def _tc1_body(deg_ref, x_ref, g3_ref):
    deg = deg_ref[0] + deg_ref[1] + 1.0
    dinv = lax.rsqrt(deg)
    g3_ref[...] = x_ref[...] * dinv[:, None]


def _tc1(deg2, xpad):
    r = NPAD // 8
    return pl.pallas_call(
        _tc1_body,
        grid=(8,),
        in_specs=[
            pl.BlockSpec((NC, r), lambda i: (0, i)),
            pl.BlockSpec((r, 8), lambda i: (i, 0)),
        ],
        out_specs=pl.BlockSpec((r, 8), lambda i: (i, 0)),
        out_shape=jax.ShapeDtypeStruct((NPAD, 8), jnp.float32),
    )(deg2, xpad)


_B2 = 1024
_G2 = NPAD // _B2


def _tc2_body(agg_ref, g3_ref, w1_ref, b1_ref, w2_ref, out_ref):
    g3 = g3_ref[...]
    a3 = agg_ref[0] + agg_ref[1] + g3
    dinv = g3[:, 3:4]
    z = a3 * dinv
    h1 = jnp.maximum(
        jnp.dot(z, w1_ref[...], preferred_element_type=jnp.float32)
        + b1_ref[...], 0.0)
    h2 = jnp.dot(h1, w2_ref[...], preferred_element_type=jnp.float32)
    out_ref[...] = h2 * dinv


def _tc2(aggA, g3p, W1p, b1r, W2):
    return pl.pallas_call(
        _tc2_body,
        grid=(_G2,),
        in_specs=[
            pl.BlockSpec((NC, _B2, 8), lambda i: (0, i, 0)),
            pl.BlockSpec((_B2, 8), lambda i: (i, 0)),
            pl.BlockSpec((8, 128), lambda i: (0, 0)),
            pl.BlockSpec((1, 128), lambda i: (0, 0)),
            pl.BlockSpec((128, 64), lambda i: (0, 0)),
        ],
        out_specs=pl.BlockSpec((_B2, 64), lambda i: (i, 0)),
        out_shape=jax.ShapeDtypeStruct((NPAD, 64), jnp.float32),
    )(aggA, g3p, W1p, b1r, W2)


def _tc3_body(agg_ref, g64_ref, g3_ref, b2_ref, wf_ref, bf_ref,
              out_ref, acc_ref):
    i = pl.program_id(0)

    @pl.when(i == 0)
    def _():
        acc_ref[...] = jnp.zeros_like(acc_ref)
        out_ref[...] = jnp.zeros_like(out_ref)

    dinv = g3_ref[:, 3:4]
    pre = dinv * (agg_ref[...] + g64_ref[...]) + b2_ref[...]
    val = jnp.maximum(pre, 0.0)
    rows = i * _B2 + lax.broadcasted_iota(jnp.int32, (_B2, 1), 0)
    val = jnp.where(rows < NNODES, val, 0.0)
    acc_ref[...] += jnp.sum(val, axis=0, keepdims=True)

    @pl.when(i == _G2 - 1)
    def _():
        mean = acc_ref[...] * (1.0 / NNODES)
        res = jnp.dot(mean, wf_ref[...], preferred_element_type=jnp.float32)
        out_ref[...] = jnp.broadcast_to(res + bf_ref[...], (8, 128))


def _tc3(agg64, g64p, g3p, b2r, Wfp, bfr):
    return pl.pallas_call(
        _tc3_body,
        grid=(_G2,),
        in_specs=[
            pl.BlockSpec((_B2, 64), lambda i: (i, 0)),
            pl.BlockSpec((_B2, 64), lambda i: (i, 0)),
            pl.BlockSpec((_B2, 8), lambda i: (i, 0)),
            pl.BlockSpec((1, 64), lambda i: (0, 0)),
            pl.BlockSpec((64, 128), lambda i: (0, 0)),
            pl.BlockSpec((1, 128), lambda i: (0, 0)),
        ],
        out_specs=pl.BlockSpec((8, 128), lambda i: (0, 0)),
        out_shape=jax.ShapeDtypeStruct((8, 128), jnp.float32),
        scratch_shapes=[pltpu.VMEM((1, 64), jnp.float32)],
    )(agg64, g64p, g3p, b2r, Wfp, bfr)


_sc_deg = _make_deg()
_sc_aggA = _make_aggA()
_sc_aggB = _make_aggB()


@jax.jit
def kernel(x, edge_index, W1, b1, W2, b2, Wf, bf):
    e0 = edge_index.shape[1]
    src = jnp.concatenate(
        [edge_index[0], jnp.full((EPAD - e0,), DUMMY, jnp.int32)]
    ).reshape(EROWS, 128)
    dst = jnp.concatenate(
        [edge_index[1], jnp.full((EPAD - e0,), DUMMY, jnp.int32)]
    ).reshape(EROWS, 128)
    xpad = (
        jnp.zeros((NPAD, 8), jnp.float32)
        .at[:NNODES, :3].set(x)
        .at[:NNODES, 3].set(1.0)
    )
    W1p = jnp.zeros((8, 128), jnp.float32).at[:3].set(W1)
    b1r = b1.reshape(1, 128)
    b2r = b2.reshape(1, 64)
    Wfp = jnp.zeros((64, 128), jnp.float32).at[:, :3].set(Wf)
    bfr = jnp.zeros((1, 128), jnp.float32).at[0, :3].set(bf)
    zD = jnp.zeros((NPAD // NS,), jnp.float32)
    zA = jnp.zeros((NPAD // NS, 8), jnp.float32)
    zB = jnp.zeros((ZCB, 64), jnp.float32)

    deg2 = _sc_deg(dst, zD).reshape(NC, NPAD)
    g3p = _tc1(deg2, xpad)
    aggA = _sc_aggA(src, dst, g3p, zA).reshape(NC, NPAD, 8)
    g64p = _tc2(aggA, g3p, W1p, b1r, W2)
    agg64 = _sc_aggB(src, dst, g64p, zB)
    outb = _tc3(agg64, g64p, g3p, b2r, Wfp, bfr)
    return outb[0, :3]


# speedup vs baseline: 1.4283x; 1.0018x over previous
import jax
import jax.numpy as jnp
from jax import lax
from jax.experimental import pallas as pl
from jax.experimental.pallas import tpu as pltpu
from jax.experimental.pallas import tpu_sc as plsc

NNODES = 50000
NPAD = 50176
EPAD = 819200
EROWS = EPAD // 128
NC, NS = 2, 16
HALF = NPAD // 2
LPAD = 25600
LDUMMY = HALF + 16
ZCB = 64
DUMMY = NNODES
KB = 8

_MESH = plsc.VectorSubcoreMesh(
    core_axis_name="c", subcore_axis_name="s", num_cores=NC, num_subcores=NS
)
_SC_PARAMS = pltpu.CompilerParams(use_tc_tiling_on_sc=False)


def _make_deg():
    def body(dst_hbm, zeros_hbm, out_hbm, dst_v, ones_v, bounce_v, deg_sh,
             sem):
        c = lax.axis_index("c")
        s = lax.axis_index("s")
        stripe = NPAD // NS
        for j in range(128 // 16):
            ones_v[pl.ds(j * 16, 16)] = jnp.ones((16,), jnp.float32)
        pltpu.sync_copy(zeros_hbm, bounce_v)
        pltpu.sync_copy(bounce_v, deg_sh.at[pl.ds(s * stripe, stripe)])
        plsc.subcore_barrier()
        rows_w = EROWS // (NC * NS)
        base = c * (EROWS // NC) + s * rows_w

        def chunk(b, carry):
            pltpu.sync_copy(dst_hbm.at[pl.ds(base + b * KB, KB)], dst_v)
            for r in range(KB):
                pltpu.sync_copy(ones_v, deg_sh.at[dst_v.at[r]], add=True)
            return carry

        lax.fori_loop(0, rows_w // KB, chunk, 0)
        plsc.subcore_barrier()
        pltpu.sync_copy(deg_sh.at[pl.ds(s * stripe, stripe)], bounce_v)
        pltpu.sync_copy(bounce_v,
                        out_hbm.at[pl.ds(c * NPAD + s * stripe, stripe)])

    return pl.kernel(
        body,
        out_type=jax.ShapeDtypeStruct((NC * NPAD,), jnp.float32),
        mesh=_MESH,
        compiler_params=_SC_PARAMS,
        scratch_types=[
            pltpu.VMEM((KB, 128), jnp.int32),
            pltpu.VMEM((128,), jnp.float32),
            pltpu.VMEM((NPAD // NS,), jnp.float32),
            pltpu.VMEM_SHARED((NPAD,), jnp.float32),
            pltpu.SemaphoreType.DMA,
        ],
    )


def _make_aggA():
    def body(src_hbm, dst_hbm, g3_hbm, zeros_hbm, out_hbm, *rest):
        src_v, dst_v = rest[0:2]
        rows = rest[2:2 + KB]
        bounce_v = rest[2 + KB]
        agg_sh = rest[3 + KB]
        sems = rest[4 + KB:4 + 2 * KB]
        c = lax.axis_index("c")
        s = lax.axis_index("s")
        stripe = NPAD // NS
        pltpu.sync_copy(zeros_hbm, bounce_v)
        pltpu.sync_copy(bounce_v, agg_sh.at[pl.ds(s * stripe, stripe)])
        plsc.subcore_barrier()
        rows_w = EROWS // (NC * NS)
        base = c * (EROWS // NC) + s * rows_w

        def chunk(b, carry):
            pltpu.sync_copy(src_hbm.at[pl.ds(base + b * KB, KB)], src_v)
            pltpu.sync_copy(dst_hbm.at[pl.ds(base + b * KB, KB)], dst_v)
            descs = [
                pltpu.async_copy(g3_hbm.at[src_v.at[r]], rows[r], sems[r])
                for r in range(KB)
            ]
            for r in range(KB):
                descs[r].wait()
                pltpu.sync_copy(rows[r], agg_sh.at[dst_v.at[r]], add=True)
            return carry

        lax.fori_loop(0, rows_w // KB, chunk, 0)
        plsc.subcore_barrier()
        pltpu.sync_copy(agg_sh.at[pl.ds(s * stripe, stripe)], bounce_v)
        pltpu.sync_copy(bounce_v,
                        out_hbm.at[pl.ds(c * NPAD + s * stripe, stripe)])

    return pl.kernel(
        body,
        out_type=jax.ShapeDtypeStruct((NC * NPAD, 8), jnp.float32),
        mesh=_MESH,
        compiler_params=_SC_PARAMS,
        scratch_types=(
            [pltpu.VMEM((KB, 128), jnp.int32)] * 2
            + [pltpu.VMEM((128, 8), jnp.float32) for _ in range(KB)]
            + [pltpu.VMEM((NPAD // NS, 8), jnp.float32),
               pltpu.VMEM_SHARED((NPAD, 8), jnp.float32)]
            + [pltpu.SemaphoreType.DMA] * KB
        ),
    )


def _make_aggB():
    NBUF = 3

    def body(src_hbm, dst_hbm, g64_hbm, zeros_hbm, out_hbm, *rest):
        src_v, dst_v, loc_v = rest[0:3]
        rows = rest[3:3 + NBUF]
        agg_sh = rest[3 + NBUF]
        sems = rest[4 + NBUF:4 + 2 * NBUF]
        c = lax.axis_index("c")
        s = lax.axis_index("s")
        zstripe = LPAD // NS
        fstripe = HALF // NS
        pltpu.sync_copy(zeros_hbm, rows[0])
        off = 0
        while off < zstripe:
            sz = min(128, zstripe - off)
            pltpu.sync_copy(rows[0].at[pl.ds(0, sz)],
                            agg_sh.at[pl.ds(s * zstripe + off, sz)])
            off += sz
        plsc.subcore_barrier()
        rows_w = EROWS // NS
        base = s * rows_w
        offset = c * HALF

        def chunk(b, carry):
            pltpu.sync_copy(src_hbm.at[pl.ds(base + b * KB, KB)], src_v)
            pltpu.sync_copy(dst_hbm.at[pl.ds(base + b * KB, KB)], dst_v)
            d = {
                r: pltpu.async_copy(g64_hbm.at[src_v.at[r]], rows[r],
                                    sems[r])
                for r in range(NBUF)
            }
            for r in range(KB):
                for k in range(128 // 16):
                    v = dst_v[r, pl.ds(k * 16, 16)]
                    loc = v - offset
                    oob = (loc < 0) | (loc >= HALF)
                    loc_v[r, pl.ds(k * 16, 16)] = jnp.where(oob, LDUMMY, loc)
            for r in range(KB):
                d[r].wait()
                pltpu.sync_copy(rows[r % NBUF], agg_sh.at[loc_v.at[r]],
                                add=True)
                if r + NBUF < KB:
                    d[r + NBUF] = pltpu.async_copy(
                        g64_hbm.at[src_v.at[r + NBUF]], rows[r % NBUF],
                        sems[r % NBUF])
            return carry

        lax.fori_loop(0, rows_w // KB, chunk, 0)
        plsc.subcore_barrier()
        off = 0
        while off < fstripe:
            sz = min(128, fstripe - off)
            pltpu.sync_copy(agg_sh.at[pl.ds(s * fstripe + off, sz)],
                            rows[0].at[pl.ds(0, sz)])
            pltpu.sync_copy(
                rows[0].at[pl.ds(0, sz)],
                out_hbm.at[pl.ds(c * HALF + s * fstripe + off, sz)])
            off += sz

    return pl.kernel(
        body,
        out_type=jax.ShapeDtypeStruct((NPAD, 64), jnp.float32),
        mesh=_MESH,
        compiler_params=_SC_PARAMS,
        scratch_types=(
            [pltpu.VMEM((KB, 128), jnp.int32)] * 3
            + [pltpu.VMEM((128, 64), jnp.float32) for _ in range(NBUF)]
            + [pltpu.VMEM_SHARED((LPAD, 64), jnp.float32)]
            + [pltpu.SemaphoreType.DMA] * NBUF
        ),
    )


def _tc1_body(deg_ref, x_ref, g3_ref):
    deg = deg_ref[0] + deg_ref[1] + 1.0
    dinv = lax.rsqrt(deg)
    g3_ref[...] = x_ref[...] * dinv[:, None]


def _tc1(deg2, xpad):
    r = NPAD // 8
    return pl.pallas_call(
        _tc1_body,
        grid=(8,),
        in_specs=[
            pl.BlockSpec((NC, r), lambda i: (0, i)),
            pl.BlockSpec((r, 8), lambda i: (i, 0)),
        ],
        out_specs=pl.BlockSpec((r, 8), lambda i: (i, 0)),
        out_shape=jax.ShapeDtypeStruct((NPAD, 8), jnp.float32),
    )(deg2, xpad)


_B2 = 1024
_G2 = NPAD // _B2


def _tc2_body(agg_ref, g3_ref, w1_ref, b1_ref, w2_ref, out_ref):
    g3 = g3_ref[...]
    a3 = agg_ref[0] + agg_ref[1] + g3
    dinv = g3[:, 3:4]
    z = a3 * dinv
    h1 = jnp.maximum(
        jnp.dot(z, w1_ref[...], preferred_element_type=jnp.float32)
        + b1_ref[...], 0.0)
    h2 = jnp.dot(h1, w2_ref[...], preferred_element_type=jnp.float32)
    out_ref[...] = h2 * dinv


def _tc2(aggA, g3p, W1p, b1r, W2):
    return pl.pallas_call(
        _tc2_body,
        grid=(_G2,),
        in_specs=[
            pl.BlockSpec((NC, _B2, 8), lambda i: (0, i, 0)),
            pl.BlockSpec((_B2, 8), lambda i: (i, 0)),
            pl.BlockSpec((8, 128), lambda i: (0, 0)),
            pl.BlockSpec((1, 128), lambda i: (0, 0)),
            pl.BlockSpec((128, 64), lambda i: (0, 0)),
        ],
        out_specs=pl.BlockSpec((_B2, 64), lambda i: (i, 0)),
        out_shape=jax.ShapeDtypeStruct((NPAD, 64), jnp.float32),
    )(aggA, g3p, W1p, b1r, W2)


def _tc3_body(agg_ref, g64_ref, g3_ref, b2_ref, wf_ref, bf_ref,
              out_ref, acc_ref):
    i = pl.program_id(0)

    @pl.when(i == 0)
    def _():
        acc_ref[...] = jnp.zeros_like(acc_ref)
        out_ref[...] = jnp.zeros_like(out_ref)

    dinv = g3_ref[:, 3:4]
    pre = dinv * (agg_ref[...] + g64_ref[...]) + b2_ref[...]
    val = jnp.maximum(pre, 0.0)
    rows = i * _B2 + lax.broadcasted_iota(jnp.int32, (_B2, 1), 0)
    val = jnp.where(rows < NNODES, val, 0.0)
    acc_ref[...] += jnp.sum(val, axis=0, keepdims=True)

    @pl.when(i == _G2 - 1)
    def _():
        mean = acc_ref[...] * (1.0 / NNODES)
        res = jnp.dot(mean, wf_ref[...], preferred_element_type=jnp.float32)
        out_ref[...] = jnp.broadcast_to(res + bf_ref[...], (8, 128))


def _tc3(agg64, g64p, g3p, b2r, Wfp, bfr):
    return pl.pallas_call(
        _tc3_body,
        grid=(_G2,),
        in_specs=[
            pl.BlockSpec((_B2, 64), lambda i: (i, 0)),
            pl.BlockSpec((_B2, 64), lambda i: (i, 0)),
            pl.BlockSpec((_B2, 8), lambda i: (i, 0)),
            pl.BlockSpec((1, 64), lambda i: (0, 0)),
            pl.BlockSpec((64, 128), lambda i: (0, 0)),
            pl.BlockSpec((1, 128), lambda i: (0, 0)),
        ],
        out_specs=pl.BlockSpec((8, 128), lambda i: (0, 0)),
        out_shape=jax.ShapeDtypeStruct((8, 128), jnp.float32),
        scratch_shapes=[pltpu.VMEM((1, 64), jnp.float32)],
    )(agg64, g64p, g3p, b2r, Wfp, bfr)


_sc_deg = _make_deg()
_sc_aggA = _make_aggA()
_sc_aggB = _make_aggB()


@jax.jit
def kernel(x, edge_index, W1, b1, W2, b2, Wf, bf):
    e0 = edge_index.shape[1]
    src = jnp.concatenate(
        [edge_index[0], jnp.full((EPAD - e0,), DUMMY, jnp.int32)]
    ).reshape(EROWS, 128)
    dst = jnp.concatenate(
        [edge_index[1], jnp.full((EPAD - e0,), DUMMY, jnp.int32)]
    ).reshape(EROWS, 128)
    xpad = (
        jnp.zeros((NPAD, 8), jnp.float32)
        .at[:NNODES, :3].set(x)
        .at[:NNODES, 3].set(1.0)
    )
    W1p = jnp.zeros((8, 128), jnp.float32).at[:3].set(W1)
    b1r = b1.reshape(1, 128)
    b2r = b2.reshape(1, 64)
    Wfp = jnp.zeros((64, 128), jnp.float32).at[:, :3].set(Wf)
    bfr = jnp.zeros((1, 128), jnp.float32).at[0, :3].set(bf)
    zD = jnp.zeros((NPAD // NS,), jnp.float32)
    zA = jnp.zeros((NPAD // NS, 8), jnp.float32)
    zB = jnp.zeros((128, 64), jnp.float32)

    deg2 = _sc_deg(dst, zD).reshape(NC, NPAD)
    g3p = _tc1(deg2, xpad)
    aggA = _sc_aggA(src, dst, g3p, zA).reshape(NC, NPAD, 8)
    g64p = _tc2(aggA, g3p, W1p, b1r, W2)
    agg64 = _sc_aggB(src, dst, g64p, zB)
    outb = _tc3(agg64, g64p, g3p, b2r, Wfp, bfr)
    return outb[0, :3]


# speedup vs baseline: 2.1220x; 1.4857x over previous
import jax
import jax.numpy as jnp
from jax import lax
from jax.experimental import pallas as pl
from jax.experimental.pallas import tpu as pltpu
from jax.experimental.pallas import tpu_sc as plsc

NNODES = 50000
NPAD = 50176
EPAD = 819200
EROWS = EPAD // 128
NC, NS = 2, 16
HALF = NPAD // 2
LPAD = 25600
LDUMMY = HALF + 16
ZCB = 64
DUMMY = NNODES
KB = 8

_MESH = plsc.VectorSubcoreMesh(
    core_axis_name="c", subcore_axis_name="s", num_cores=NC, num_subcores=NS
)
_SC_PARAMS = pltpu.CompilerParams(use_tc_tiling_on_sc=False)


def _make_deg():
    def body(dst_hbm, zeros_hbm, out_hbm, dst_v, ones_v, bounce_v, deg_sh,
             sem):
        c = lax.axis_index("c")
        s = lax.axis_index("s")
        stripe = NPAD // NS
        for j in range(128 // 16):
            ones_v[pl.ds(j * 16, 16)] = jnp.ones((16,), jnp.float32)
        pltpu.sync_copy(zeros_hbm, bounce_v)
        pltpu.sync_copy(bounce_v, deg_sh.at[pl.ds(s * stripe, stripe)])
        plsc.subcore_barrier()
        rows_w = EROWS // (NC * NS)
        base = c * (EROWS // NC) + s * rows_w

        def chunk(b, carry):
            pltpu.sync_copy(dst_hbm.at[pl.ds(base + b * KB, KB)], dst_v)
            for r in range(KB):
                pltpu.sync_copy(ones_v, deg_sh.at[dst_v.at[r]], add=True)
            return carry

        lax.fori_loop(0, rows_w // KB, chunk, 0)
        plsc.subcore_barrier()
        pltpu.sync_copy(deg_sh.at[pl.ds(s * stripe, stripe)], bounce_v)
        pltpu.sync_copy(bounce_v,
                        out_hbm.at[pl.ds(c * NPAD + s * stripe, stripe)])

    return pl.kernel(
        body,
        out_type=jax.ShapeDtypeStruct((NC * NPAD,), jnp.float32),
        mesh=_MESH,
        compiler_params=_SC_PARAMS,
        scratch_types=[
            pltpu.VMEM((KB, 128), jnp.int32),
            pltpu.VMEM((128,), jnp.float32),
            pltpu.VMEM((NPAD // NS,), jnp.float32),
            pltpu.VMEM_SHARED((NPAD,), jnp.float32),
            pltpu.SemaphoreType.DMA,
        ],
    )


def _make_aggA():
    def body(src_hbm, dst_hbm, g3_hbm, zeros_hbm, out_hbm, *rest):
        src_v, dst_v = rest[0:2]
        rows = rest[2:2 + KB]
        bounce_v = rest[2 + KB]
        agg_sh = rest[3 + KB]
        sems = rest[4 + KB:4 + 2 * KB]
        c = lax.axis_index("c")
        s = lax.axis_index("s")
        stripe = NPAD // NS
        pltpu.sync_copy(zeros_hbm, bounce_v)
        pltpu.sync_copy(bounce_v, agg_sh.at[pl.ds(s * stripe, stripe)])
        plsc.subcore_barrier()
        rows_w = EROWS // (NC * NS)
        base = c * (EROWS // NC) + s * rows_w

        def chunk(b, carry):
            pltpu.sync_copy(src_hbm.at[pl.ds(base + b * KB, KB)], src_v)
            pltpu.sync_copy(dst_hbm.at[pl.ds(base + b * KB, KB)], dst_v)
            descs = [
                pltpu.async_copy(g3_hbm.at[src_v.at[r]], rows[r], sems[r])
                for r in range(KB)
            ]
            for r in range(KB):
                descs[r].wait()
                pltpu.sync_copy(rows[r], agg_sh.at[dst_v.at[r]], add=True)
            return carry

        lax.fori_loop(0, rows_w // KB, chunk, 0)
        plsc.subcore_barrier()
        pltpu.sync_copy(agg_sh.at[pl.ds(s * stripe, stripe)], bounce_v)
        pltpu.sync_copy(bounce_v,
                        out_hbm.at[pl.ds(c * NPAD + s * stripe, stripe)])

    return pl.kernel(
        body,
        out_type=jax.ShapeDtypeStruct((NC * NPAD, 8), jnp.float32),
        mesh=_MESH,
        compiler_params=_SC_PARAMS,
        scratch_types=(
            [pltpu.VMEM((KB, 128), jnp.int32)] * 2
            + [pltpu.VMEM((128, 8), jnp.float32) for _ in range(KB)]
            + [pltpu.VMEM((NPAD // NS, 8), jnp.float32),
               pltpu.VMEM_SHARED((NPAD, 8), jnp.float32)]
            + [pltpu.SemaphoreType.DMA] * KB
        ),
    )


def _make_aggB():
    NBUF = 3

    def body(src_hbm, dst_hbm, gcat_hbm, zeros_hbm, out_hbm, *rest):
        src_v, dst_v, loc_v = rest[0:3]
        rows = rest[3:3 + NBUF]
        agg_sh = rest[3 + NBUF]
        sems = rest[4 + NBUF:4 + 2 * NBUF]
        c = lax.axis_index("c")
        s = lax.axis_index("s")
        stripe = NPAD // NS
        pltpu.sync_copy(zeros_hbm, rows[0])
        off = 0
        while off < stripe:
            sz = min(128, stripe - off)
            pltpu.sync_copy(rows[0].at[pl.ds(0, sz)],
                            agg_sh.at[pl.ds(s * stripe + off, sz)])
            off += sz
        plsc.subcore_barrier()
        rows_w = EROWS // NS
        base = s * rows_w
        offset = c * NPAD

        def chunk(b, carry):
            pltpu.sync_copy(src_hbm.at[pl.ds(base + b * KB, KB)], src_v)
            pltpu.sync_copy(dst_hbm.at[pl.ds(base + b * KB, KB)], dst_v)
            for r in range(KB):
                for k in range(128 // 16):
                    loc_v[r, pl.ds(k * 16, 16)] = (
                        src_v[r, pl.ds(k * 16, 16)] + offset)
            d = {
                r: pltpu.async_copy(gcat_hbm.at[loc_v.at[r]], rows[r],
                                    sems[r])
                for r in range(NBUF)
            }
            for r in range(KB):
                d[r].wait()
                pltpu.sync_copy(rows[r % NBUF], agg_sh.at[dst_v.at[r]],
                                add=True)
                if r + NBUF < KB:
                    d[r + NBUF] = pltpu.async_copy(
                        gcat_hbm.at[loc_v.at[r + NBUF]], rows[r % NBUF],
                        sems[r % NBUF])
            return carry

        lax.fori_loop(0, rows_w // KB, chunk, 0)
        plsc.subcore_barrier()
        off = 0
        while off < stripe:
            sz = min(128, stripe - off)
            pltpu.sync_copy(agg_sh.at[pl.ds(s * stripe + off, sz)],
                            rows[0].at[pl.ds(0, sz)])
            pltpu.sync_copy(
                rows[0].at[pl.ds(0, sz)],
                out_hbm.at[pl.ds(c * NPAD + s * stripe + off, sz)])
            off += sz

    return pl.kernel(
        body,
        out_type=jax.ShapeDtypeStruct((NC * NPAD, 32), jnp.float32),
        mesh=_MESH,
        compiler_params=_SC_PARAMS,
        scratch_types=(
            [pltpu.VMEM((KB, 128), jnp.int32)] * 3
            + [pltpu.VMEM((128, 32), jnp.float32) for _ in range(NBUF)]
            + [pltpu.VMEM_SHARED((NPAD, 32), jnp.float32)]
            + [pltpu.SemaphoreType.DMA] * NBUF
        ),
    )


def _tc1_body(deg_ref, x_ref, g3_ref):
    deg = deg_ref[0] + deg_ref[1] + 1.0
    dinv = lax.rsqrt(deg)
    g3_ref[...] = x_ref[...] * dinv[:, None]


def _tc1(deg2, xpad):
    r = NPAD // 8
    return pl.pallas_call(
        _tc1_body,
        grid=(8,),
        in_specs=[
            pl.BlockSpec((NC, r), lambda i: (0, i)),
            pl.BlockSpec((r, 8), lambda i: (i, 0)),
        ],
        out_specs=pl.BlockSpec((r, 8), lambda i: (i, 0)),
        out_shape=jax.ShapeDtypeStruct((NPAD, 8), jnp.float32),
    )(deg2, xpad)


_B2 = 1024
_G2 = NPAD // _B2


def _tc2_body(agg_ref, g3_ref, w1_ref, b1_ref, w2_ref, out_ref):
    g3 = g3_ref[...]
    a3 = agg_ref[0] + agg_ref[1] + g3
    dinv = g3[:, 3:4]
    z = a3 * dinv
    h1 = jnp.maximum(
        jnp.dot(z, w1_ref[...], preferred_element_type=jnp.float32)
        + b1_ref[...], 0.0)
    h2 = jnp.dot(h1, w2_ref[...], preferred_element_type=jnp.float32)
    g64 = h2 * dinv
    out_ref[0, ...] = g64[:, :32]
    out_ref[1, ...] = g64[:, 32:]


def _tc2(aggA, g3p, W1p, b1r, W2):
    return pl.pallas_call(
        _tc2_body,
        grid=(_G2,),
        in_specs=[
            pl.BlockSpec((NC, _B2, 8), lambda i: (0, i, 0)),
            pl.BlockSpec((_B2, 8), lambda i: (i, 0)),
            pl.BlockSpec((8, 128), lambda i: (0, 0)),
            pl.BlockSpec((1, 128), lambda i: (0, 0)),
            pl.BlockSpec((128, 64), lambda i: (0, 0)),
        ],
        out_specs=pl.BlockSpec((NC, _B2, 32), lambda i: (0, i, 0)),
        out_shape=jax.ShapeDtypeStruct((NC, NPAD, 32), jnp.float32),
    )(aggA, g3p, W1p, b1r, W2)


def _tc3_body(agg_ref, g64_ref, g3_ref, b2_ref, wf_ref, bf_ref,
              out_ref, acc_ref):
    i = pl.program_id(0)

    @pl.when(i == 0)
    def _():
        acc_ref[...] = jnp.zeros_like(acc_ref)
        out_ref[...] = jnp.zeros_like(out_ref)

    dinv = g3_ref[:, 3:4]
    rows = i * _B2 + lax.broadcasted_iota(jnp.int32, (_B2, 1), 0)
    valid = rows < NNODES
    for h in range(NC):
        pre = (dinv * (agg_ref[h] + g64_ref[h])
               + b2_ref[:, 32 * h:32 * h + 32])
        val = jnp.where(valid, jnp.maximum(pre, 0.0), 0.0)
        acc_ref[:, 32 * h:32 * h + 32] += jnp.sum(val, axis=0, keepdims=True)

    @pl.when(i == _G2 - 1)
    def _():
        mean = acc_ref[...] * (1.0 / NNODES)
        res = jnp.dot(mean, wf_ref[...], preferred_element_type=jnp.float32)
        out_ref[...] = jnp.broadcast_to(res + bf_ref[...], (8, 128))


def _tc3(agg64s, g64s, g3p, b2r, Wfp, bfr):
    return pl.pallas_call(
        _tc3_body,
        grid=(_G2,),
        in_specs=[
            pl.BlockSpec((NC, _B2, 32), lambda i: (0, i, 0)),
            pl.BlockSpec((NC, _B2, 32), lambda i: (0, i, 0)),
            pl.BlockSpec((_B2, 8), lambda i: (i, 0)),
            pl.BlockSpec((1, 64), lambda i: (0, 0)),
            pl.BlockSpec((64, 128), lambda i: (0, 0)),
            pl.BlockSpec((1, 128), lambda i: (0, 0)),
        ],
        out_specs=pl.BlockSpec((8, 128), lambda i: (0, 0)),
        out_shape=jax.ShapeDtypeStruct((8, 128), jnp.float32),
        scratch_shapes=[pltpu.VMEM((1, 64), jnp.float32)],
    )(agg64s, g64s, g3p, b2r, Wfp, bfr)


_sc_deg = _make_deg()
_sc_aggA = _make_aggA()
_sc_aggB = _make_aggB()


@jax.jit
def kernel(x, edge_index, W1, b1, W2, b2, Wf, bf):
    e0 = edge_index.shape[1]
    src = jnp.concatenate(
        [edge_index[0], jnp.full((EPAD - e0,), DUMMY, jnp.int32)]
    ).reshape(EROWS, 128)
    dst = jnp.concatenate(
        [edge_index[1], jnp.full((EPAD - e0,), DUMMY, jnp.int32)]
    ).reshape(EROWS, 128)
    xpad = (
        jnp.zeros((NPAD, 8), jnp.float32)
        .at[:NNODES, :3].set(x)
        .at[:NNODES, 3].set(1.0)
    )
    W1p = jnp.zeros((8, 128), jnp.float32).at[:3].set(W1)
    b1r = b1.reshape(1, 128)
    b2r = b2.reshape(1, 64)
    Wfp = jnp.zeros((64, 128), jnp.float32).at[:, :3].set(Wf)
    bfr = jnp.zeros((1, 128), jnp.float32).at[0, :3].set(bf)
    zD = jnp.zeros((NPAD // NS,), jnp.float32)
    zA = jnp.zeros((NPAD // NS, 8), jnp.float32)
    zB = jnp.zeros((128, 32), jnp.float32)

    deg2 = _sc_deg(dst, zD).reshape(NC, NPAD)
    g3p = _tc1(deg2, xpad)
    aggA = _sc_aggA(src, dst, g3p, zA).reshape(NC, NPAD, 8)
    g64s = _tc2(aggA, g3p, W1p, b1r, W2)
    gcat = g64s.reshape(NC * NPAD, 32)
    agg64s = _sc_aggB(src, dst, gcat, zB).reshape(NC, NPAD, 32)
    outb = _tc3(agg64s, g64s, g3p, b2r, Wfp, bfr)
    return outb[0, :3]
